# Initial kernel scaffold; baseline (speedup 1.0000x reference)
import jax
import jax.numpy as jnp
from jax.experimental import pallas as pl


def kernel(A, X, A2, X2, W11, v111, v112, W12, v121, v122, W21, v211, v212, W22, v221, v222, mu):
    raise NotImplementedError("write your pallas kernel here")


# speedup vs baseline: 1.5506x; 1.5506x over previous
import functools

import jax
import jax.numpy as jnp
from jax.experimental import pallas as pl
from jax.experimental.pallas import tpu as pltpu

_N = 4096
_TM = 256
_BETA = 0.5
_ALPHA = 1.0


def _unnorm_attn(A_t, a_col, b_row):
    logits = A_t * a_col + A_t * b_row
    mask = logits != 0.0
    return jnp.where(mask, jnp.exp(jax.nn.sigmoid(logits)), 0.0)


def _proj_kernel(x_ref, w_ref, v1_ref, v2_ref, h_ref, a_ref, b_ref):
    h = jnp.dot(x_ref[...], w_ref[...], preferred_element_type=jnp.float32)
    h_ref[...] = h
    a_ref[...] = jnp.dot(h, v1_ref[...], preferred_element_type=jnp.float32)
    b_ref[...] = jnp.dot(h, v2_ref[...], preferred_element_type=jnp.float32)


def _proj(X, W1, v11, v12):
    k_in, k_out = W1.shape
    tm = 512
    return pl.pallas_call(
        _proj_kernel,
        grid=(_N // tm,),
        in_specs=[
            pl.BlockSpec((tm, k_in), lambda i: (i, 0)),
            pl.BlockSpec((k_in, k_out), lambda i: (0, 0)),
            pl.BlockSpec((k_out, 1), lambda i: (0, 0)),
            pl.BlockSpec((k_out, 1), lambda i: (0, 0)),
        ],
        out_specs=[
            pl.BlockSpec((tm, k_out), lambda i: (i, 0)),
            pl.BlockSpec((tm, 1), lambda i: (i, 0)),
            pl.BlockSpec((tm, 1), lambda i: (i, 0)),
        ],
        out_shape=[
            jax.ShapeDtypeStruct((_N, k_out), jnp.float32),
            jax.ShapeDtypeStruct((_N, 1), jnp.float32),
            jax.ShapeDtypeStruct((_N, 1), jnp.float32),
        ],
        compiler_params=pltpu.CompilerParams(dimension_semantics=("parallel",)),
    )(X, W1, v11, v12)


def _gat_enc_kernel(with_ab, transpose_w, a_ref, b_ref, A_ref, h_ref, w_ref,
                    *rest):
    if with_ab:
        v1_ref, v2_ref, S_ref, r_ref, hout_ref, aout_ref, bout_ref = rest
        o_ref = None
    else:
        S_ref, r_ref, o_ref, hout_ref = rest
    s = _unnorm_attn(A_ref[...], a_ref[...], b_ref[...])
    S_ref[...] = s.astype(jnp.bfloat16)
    r = jnp.sum(s, axis=1, keepdims=True)
    r_ref[...] = r
    o = jnp.dot(s.astype(jnp.bfloat16), h_ref[...].astype(jnp.bfloat16),
                preferred_element_type=jnp.float32)
    o = o / jnp.where(r == 0.0, 1.0, r)
    if o_ref is not None:
        o_ref[...] = o
    if transpose_w:
        ho = jax.lax.dot_general(o, w_ref[...], (((1,), (1,)), ((), ())),
                                 preferred_element_type=jnp.float32)
    else:
        ho = jnp.dot(o, w_ref[...], preferred_element_type=jnp.float32)
    hout_ref[...] = ho
    if with_ab:
        aout_ref[...] = jnp.dot(ho, v1_ref[...],
                                preferred_element_type=jnp.float32)
        bout_ref[...] = jnp.dot(ho, v2_ref[...],
                                preferred_element_type=jnp.float32)


def _gat_enc1(A, a, b, H, W, v1, v2):
    k = H.shape[1]
    k2 = W.shape[1]
    body = functools.partial(_gat_enc_kernel, True, False)
    return pl.pallas_call(
        body,
        grid=(_N // _TM,),
        in_specs=[
            pl.BlockSpec((_TM, 1), lambda i: (i, 0)),
            pl.BlockSpec((1, _N), lambda i: (0, 0)),
            pl.BlockSpec((_TM, _N), lambda i: (i, 0)),
            pl.BlockSpec((_N, k), lambda i: (0, 0)),
            pl.BlockSpec((k, k2), lambda i: (0, 0)),
            pl.BlockSpec((k2, 1), lambda i: (0, 0)),
            pl.BlockSpec((k2, 1), lambda i: (0, 0)),
        ],
        out_specs=[
            pl.BlockSpec((_TM, _N), lambda i: (i, 0)),
            pl.BlockSpec((_TM, 1), lambda i: (i, 0)),
            pl.BlockSpec((_TM, k2), lambda i: (i, 0)),
            pl.BlockSpec((_TM, 1), lambda i: (i, 0)),
            pl.BlockSpec((_TM, 1), lambda i: (i, 0)),
        ],
        out_shape=[
            jax.ShapeDtypeStruct((_N, _N), jnp.bfloat16),
            jax.ShapeDtypeStruct((_N, 1), jnp.float32),
            jax.ShapeDtypeStruct((_N, k2), jnp.float32),
            jax.ShapeDtypeStruct((_N, 1), jnp.float32),
            jax.ShapeDtypeStruct((_N, 1), jnp.float32),
        ],
        compiler_params=pltpu.CompilerParams(dimension_semantics=("parallel",)),
    )(a, b, A, H, W, v1, v2)


def _gat_enc2(A, a, b, H, W):
    k = H.shape[1]
    k2 = W.shape[0]
    body = functools.partial(_gat_enc_kernel, False, True)
    return pl.pallas_call(
        body,
        grid=(_N // _TM,),
        in_specs=[
            pl.BlockSpec((_TM, 1), lambda i: (i, 0)),
            pl.BlockSpec((1, _N), lambda i: (0, 0)),
            pl.BlockSpec((_TM, _N), lambda i: (i, 0)),
            pl.BlockSpec((_N, k), lambda i: (0, 0)),
            pl.BlockSpec((k2, k), lambda i: (0, 0)),
        ],
        out_specs=[
            pl.BlockSpec((_TM, _N), lambda i: (i, 0)),
            pl.BlockSpec((_TM, 1), lambda i: (i, 0)),
            pl.BlockSpec((_TM, k), lambda i: (i, 0)),
            pl.BlockSpec((_TM, k2), lambda i: (i, 0)),
        ],
        out_shape=[
            jax.ShapeDtypeStruct((_N, _N), jnp.bfloat16),
            jax.ShapeDtypeStruct((_N, 1), jnp.float32),
            jax.ShapeDtypeStruct((_N, k), jnp.float32),
            jax.ShapeDtypeStruct((_N, k2), jnp.float32),
        ],
        compiler_params=pltpu.CompilerParams(dimension_semantics=("parallel",)),
    )(a, b, A, H, W)


def _gat_dec_kernel(with_w, S_ref, r_ref, h_ref, *rest):
    if with_w:
        w_ref, out_ref = rest
    else:
        out_ref, = rest
    o = jnp.dot(S_ref[...], h_ref[...].astype(jnp.bfloat16),
                preferred_element_type=jnp.float32)
    r = r_ref[...]
    o = o / jnp.where(r == 0.0, 1.0, r)
    if with_w:
        o = jax.lax.dot_general(o, w_ref[...], (((1,), (1,)), ((), ())),
                                preferred_element_type=jnp.float32)
    out_ref[...] = o


def _gat_dec(S, r, H, W=None):
    k = H.shape[1]
    kout = W.shape[0] if W is not None else k
    in_specs = [
        pl.BlockSpec((_TM, _N), lambda i: (i, 0)),
        pl.BlockSpec((_TM, 1), lambda i: (i, 0)),
        pl.BlockSpec((_N, k), lambda i: (0, 0)),
    ]
    args = [S, r, H]
    if W is not None:
        in_specs.append(pl.BlockSpec((kout, k), lambda i: (0, 0)))
        args.append(W)
    body = functools.partial(_gat_dec_kernel, W is not None)
    return pl.pallas_call(
        body,
        grid=(_N // _TM,),
        in_specs=in_specs,
        out_specs=pl.BlockSpec((_TM, kout), lambda i: (i, 0)),
        out_shape=jax.ShapeDtypeStruct((_N, kout), jnp.float32),
        compiler_params=pltpu.CompilerParams(dimension_semantics=("parallel",)),
    )(*args)


def _final_kernel(h1_ref, h2_ref, mu_ref, hf_ref, q_ref):
    hf = h1_ref[...] + _BETA * h2_ref[...]
    hf_ref[...] = hf
    mu = mu_ref[...]
    hn = jnp.sum(hf * hf, axis=1, keepdims=True)
    mn = jnp.sum(mu * mu, axis=1)[None, :]
    cross = jax.lax.dot_general(hf, mu, (((1,), (1,)), ((), ())),
                                preferred_element_type=jnp.float32)
    d2 = hn + mn - 2.0 * cross
    qun = (1.0 + d2 / _ALPHA) ** (-(_ALPHA + 1.0) / 2.0)
    q_ref[...] = qun / jnp.sum(qun, axis=1, keepdims=True)


def _final(H1, H2, mu):
    k = H1.shape[1]
    nc = mu.shape[0]
    tm = 512
    return pl.pallas_call(
        _final_kernel,
        grid=(_N // tm,),
        in_specs=[
            pl.BlockSpec((tm, k), lambda i: (i, 0)),
            pl.BlockSpec((tm, k), lambda i: (i, 0)),
            pl.BlockSpec((nc, k), lambda i: (0, 0)),
        ],
        out_specs=[
            pl.BlockSpec((tm, k), lambda i: (i, 0)),
            pl.BlockSpec((tm, nc), lambda i: (i, 0)),
        ],
        out_shape=[
            jax.ShapeDtypeStruct((_N, k), jnp.float32),
            jax.ShapeDtypeStruct((_N, nc), jnp.float32),
        ],
        compiler_params=pltpu.CompilerParams(dimension_semantics=("parallel",)),
    )(H1, H2, mu)


def _branch(A, X, W1, v11, v12, W2, v21, v22):
    H0, a0, b0 = _proj(X, W1, v11, v12)
    b0r = b0.reshape(1, _N)
    S0, r0, H1, a1, b1 = _gat_enc1(A, a0, b0r, H0, W2, v21, v22)
    b1r = b1.reshape(1, _N)
    S1, r1, emb, D1 = _gat_enc2(A, a1, b1r, H1, W2)
    D0 = _gat_dec(S1, r1, D1, W1)
    X_ = _gat_dec(S0, r0, D0)
    return emb, X_


def kernel(A, X, A2, X2, W11, v111, v112, W12, v121, v122, W21, v211, v212,
           W22, v221, v222, mu):
    H1, X_ = _branch(A, X, W11, v111, v112, W12, v121, v122)
    H2, X_2 = _branch(A2, X2, W21, v211, v212, W22, v221, v222)
    H_F, q = _final(H1, H2, mu)
    return (H_F, q, H1, H2, X_, X_2)


# speedup vs baseline: 1.8062x; 1.1649x over previous
import functools

import jax
import jax.numpy as jnp
from jax.experimental import pallas as pl
from jax.experimental.pallas import tpu as pltpu

_N = 4096
_TM = 512
_BETA = 0.5
_ALPHA = 1.0
_C = 0.72134752044448170367996234050095


def _unnorm_attn(A_t, ah_col, bh_row):
    half = A_t * ah_col + A_t * bh_row
    s = jnp.exp2(_C * jnp.tanh(half) + _C)
    return jnp.where(half != 0.0, s, 0.0).astype(jnp.bfloat16)


def _ones_aug(h, tm):
    return jnp.concatenate([h, jnp.ones((tm, 1), jnp.float32)],
                           axis=1).astype(jnp.bfloat16)


def _proj_kernel(x_ref, w_ref, v1_ref, v2_ref, haug_ref, a_ref, b_ref):
    h = jnp.dot(x_ref[...], w_ref[...], preferred_element_type=jnp.float32)
    haug_ref[...] = _ones_aug(h, h.shape[0])
    a_ref[...] = 0.5 * jnp.dot(h, v1_ref[...],
                               preferred_element_type=jnp.float32)
    b_ref[...] = 0.5 * jnp.dot(h, v2_ref[...],
                               preferred_element_type=jnp.float32)


def _proj(X, W1, v11, v12):
    k_in, k_out = W1.shape
    tm = 512
    return pl.pallas_call(
        _proj_kernel,
        grid=(_N // tm,),
        in_specs=[
            pl.BlockSpec((tm, k_in), lambda i: (i, 0)),
            pl.BlockSpec((k_in, k_out), lambda i: (0, 0)),
            pl.BlockSpec((k_out, 1), lambda i: (0, 0)),
            pl.BlockSpec((k_out, 1), lambda i: (0, 0)),
        ],
        out_specs=[
            pl.BlockSpec((tm, k_out + 1), lambda i: (i, 0)),
            pl.BlockSpec((tm, 1), lambda i: (i, 0)),
            pl.BlockSpec((tm, 1), lambda i: (i, 0)),
        ],
        out_shape=[
            jax.ShapeDtypeStruct((_N, k_out + 1), jnp.bfloat16),
            jax.ShapeDtypeStruct((_N, 1), jnp.float32),
            jax.ShapeDtypeStruct((_N, 1), jnp.float32),
        ],
        compiler_params=pltpu.CompilerParams(dimension_semantics=("parallel",)),
    )(X, W1, v11, v12)


def _recip_r(oa, k):
    r = oa[:, k:k + 1]
    return r, 1.0 / jnp.where(r == 0.0, 1.0, r)


def _gat_enc1_kernel(a_ref, b_ref, A_ref, haug_ref, w_ref, v1_ref, v2_ref,
                     S_ref, r_ref, haug_out_ref, aout_ref, bout_ref):
    k = haug_ref.shape[1] - 1
    sb = _unnorm_attn(A_ref[...], a_ref[...], b_ref[...])
    S_ref[...] = sb
    oa = jnp.dot(sb, haug_ref[...], preferred_element_type=jnp.float32)
    r, rinv = _recip_r(oa, k)
    r_ref[...] = r
    o = oa[:, :k] * rinv
    ho = jnp.dot(o, w_ref[...], preferred_element_type=jnp.float32)
    haug_out_ref[...] = _ones_aug(ho, ho.shape[0])
    aout_ref[...] = 0.5 * jnp.dot(ho, v1_ref[...],
                                  preferred_element_type=jnp.float32)
    bout_ref[...] = 0.5 * jnp.dot(ho, v2_ref[...],
                                  preferred_element_type=jnp.float32)


def _gat_enc1(A, a, b, Haug, W, v1, v2):
    ka = Haug.shape[1]
    k2 = W.shape[1]
    return pl.pallas_call(
        _gat_enc1_kernel,
        grid=(_N // _TM,),
        in_specs=[
            pl.BlockSpec((_TM, 1), lambda i: (i, 0)),
            pl.BlockSpec((1, _N), lambda i: (0, 0)),
            pl.BlockSpec((_TM, _N), lambda i: (i, 0)),
            pl.BlockSpec((_N, ka), lambda i: (0, 0)),
            pl.BlockSpec((ka - 1, k2), lambda i: (0, 0)),
            pl.BlockSpec((k2, 1), lambda i: (0, 0)),
            pl.BlockSpec((k2, 1), lambda i: (0, 0)),
        ],
        out_specs=[
            pl.BlockSpec((_TM, _N), lambda i: (i, 0)),
            pl.BlockSpec((_TM, 1), lambda i: (i, 0)),
            pl.BlockSpec((_TM, k2 + 1), lambda i: (i, 0)),
            pl.BlockSpec((_TM, 1), lambda i: (i, 0)),
            pl.BlockSpec((_TM, 1), lambda i: (i, 0)),
        ],
        out_shape=[
            jax.ShapeDtypeStruct((_N, _N), jnp.bfloat16),
            jax.ShapeDtypeStruct((_N, 1), jnp.float32),
            jax.ShapeDtypeStruct((_N, k2 + 1), jnp.bfloat16),
            jax.ShapeDtypeStruct((_N, 1), jnp.float32),
            jax.ShapeDtypeStruct((_N, 1), jnp.float32),
        ],
        compiler_params=pltpu.CompilerParams(dimension_semantics=("parallel",)),
    )(a, b, A, Haug, W, v1, v2)


def _gat_enc2_kernel(a_ref, b_ref, A_ref, haug_ref, w_ref,
                     S_ref, r_ref, emb_ref, d_ref):
    k = haug_ref.shape[1] - 1
    sb = _unnorm_attn(A_ref[...], a_ref[...], b_ref[...])
    S_ref[...] = sb
    oa = jnp.dot(sb, haug_ref[...], preferred_element_type=jnp.float32)
    r, rinv = _recip_r(oa, k)
    r_ref[...] = r
    o = oa[:, :k] * rinv
    emb_ref[...] = o
    d = jax.lax.dot_general(o, w_ref[...], (((1,), (1,)), ((), ())),
                            preferred_element_type=jnp.float32)
    d_ref[...] = d.astype(jnp.bfloat16)


def _gat_enc2(A, a, b, Haug, W):
    ka = Haug.shape[1]
    k2 = W.shape[0]
    return pl.pallas_call(
        _gat_enc2_kernel,
        grid=(_N // _TM,),
        in_specs=[
            pl.BlockSpec((_TM, 1), lambda i: (i, 0)),
            pl.BlockSpec((1, _N), lambda i: (0, 0)),
            pl.BlockSpec((_TM, _N), lambda i: (i, 0)),
            pl.BlockSpec((_N, ka), lambda i: (0, 0)),
            pl.BlockSpec((k2, ka - 1), lambda i: (0, 0)),
        ],
        out_specs=[
            pl.BlockSpec((_TM, _N), lambda i: (i, 0)),
            pl.BlockSpec((_TM, 1), lambda i: (i, 0)),
            pl.BlockSpec((_TM, ka - 1), lambda i: (i, 0)),
            pl.BlockSpec((_TM, k2), lambda i: (i, 0)),
        ],
        out_shape=[
            jax.ShapeDtypeStruct((_N, _N), jnp.bfloat16),
            jax.ShapeDtypeStruct((_N, 1), jnp.float32),
            jax.ShapeDtypeStruct((_N, ka - 1), jnp.float32),
            jax.ShapeDtypeStruct((_N, k2), jnp.bfloat16),
        ],
        compiler_params=pltpu.CompilerParams(dimension_semantics=("parallel",)),
    )(a, b, A, Haug, W)


def _gat_dec_kernel(with_w, S_ref, r_ref, h_ref, *rest):
    if with_w:
        w_ref, out_ref = rest
    else:
        out_ref, = rest
    o = jnp.dot(S_ref[...], h_ref[...], preferred_element_type=jnp.float32)
    r = r_ref[...]
    o = o * (1.0 / jnp.where(r == 0.0, 1.0, r))
    if with_w:
        out_ref[...] = jax.lax.dot_general(
            o, w_ref[...], (((1,), (1,)), ((), ())),
            preferred_element_type=jnp.float32)
    else:
        out_ref[...] = o.astype(out_ref.dtype)


def _gat_dec(S, r, H, W=None, out_dtype=jnp.float32):
    k = H.shape[1]
    kout = W.shape[0] if W is not None else k
    in_specs = [
        pl.BlockSpec((_TM, _N), lambda i: (i, 0)),
        pl.BlockSpec((_TM, 1), lambda i: (i, 0)),
        pl.BlockSpec((_N, k), lambda i: (0, 0)),
    ]
    args = [S, r, H]
    if W is not None:
        in_specs.append(pl.BlockSpec((kout, k), lambda i: (0, 0)))
        args.append(W)
        out_dtype = jnp.float32
    body = functools.partial(_gat_dec_kernel, W is not None)
    return pl.pallas_call(
        body,
        grid=(_N // _TM,),
        in_specs=in_specs,
        out_specs=pl.BlockSpec((_TM, kout), lambda i: (i, 0)),
        out_shape=jax.ShapeDtypeStruct((_N, kout), out_dtype),
        compiler_params=pltpu.CompilerParams(dimension_semantics=("parallel",)),
    )(*args)


def _final_kernel(h1_ref, h2_ref, mu_ref, hf_ref, q_ref):
    hf = h1_ref[...] + _BETA * h2_ref[...]
    hf_ref[...] = hf
    mu = mu_ref[...]
    hn = jnp.sum(hf * hf, axis=1, keepdims=True)
    mn = jnp.sum(mu * mu, axis=1)[None, :]
    cross = jax.lax.dot_general(hf, mu, (((1,), (1,)), ((), ())),
                                preferred_element_type=jnp.float32)
    d2 = hn + mn - 2.0 * cross
    qun = (1.0 + d2 / _ALPHA) ** (-(_ALPHA + 1.0) / 2.0)
    q_ref[...] = qun / jnp.sum(qun, axis=1, keepdims=True)


def _final(H1, H2, mu):
    k = H1.shape[1]
    nc = mu.shape[0]
    tm = 512
    return pl.pallas_call(
        _final_kernel,
        grid=(_N // tm,),
        in_specs=[
            pl.BlockSpec((tm, k), lambda i: (i, 0)),
            pl.BlockSpec((tm, k), lambda i: (i, 0)),
            pl.BlockSpec((nc, k), lambda i: (0, 0)),
        ],
        out_specs=[
            pl.BlockSpec((tm, k), lambda i: (i, 0)),
            pl.BlockSpec((tm, nc), lambda i: (i, 0)),
        ],
        out_shape=[
            jax.ShapeDtypeStruct((_N, k), jnp.float32),
            jax.ShapeDtypeStruct((_N, nc), jnp.float32),
        ],
        compiler_params=pltpu.CompilerParams(dimension_semantics=("parallel",)),
    )(H1, H2, mu)


def _branch(A, X, W1, v11, v12, W2, v21, v22):
    H0aug, a0, b0 = _proj(X, W1, v11, v12)
    S0, r0, H1aug, a1, b1 = _gat_enc1(A, a0, b0.reshape(1, _N), H0aug,
                                      W2, v21, v22)
    S1, r1, emb, D1 = _gat_enc2(A, a1, b1.reshape(1, _N), H1aug, W2)
    T = _gat_dec(S1, r1, D1, out_dtype=jnp.bfloat16)
    X_ = _gat_dec(S0, r0, T, W1)
    return emb, X_


def kernel(A, X, A2, X2, W11, v111, v112, W12, v121, v122, W21, v211, v212,
           W22, v221, v222, mu):
    H1, X_ = _branch(A, X, W11, v111, v112, W12, v121, v122)
    H2, X_2 = _branch(A2, X2, W21, v211, v212, W22, v221, v222)
    H_F, q = _final(H1, H2, mu)
    return (H_F, q, H1, H2, X_, X_2)


# speedup vs baseline: 1.8600x; 1.0297x over previous
import functools

import jax
import jax.numpy as jnp
from jax.experimental import pallas as pl
from jax.experimental.pallas import tpu as pltpu

_N = 4096
_TM = 512
_BETA = 0.5
_ALPHA = 1.0
_C = 0.72134752044448170367996234050095


def _unnorm_attn(A_t, ah_col, bh_row):
    half = A_t.astype(jnp.bfloat16) * (ah_col + bh_row)
    cb = jnp.bfloat16(_C)
    s = jnp.exp2(cb * jnp.tanh(half) + cb)
    return jnp.where(half != jnp.bfloat16(0.0), s, jnp.bfloat16(0.0))


def _ones_aug(h, tm):
    return jnp.concatenate([h, jnp.ones((tm, 1), jnp.float32)],
                           axis=1).astype(jnp.bfloat16)


def _proj_kernel(x_ref, w_ref, v1_ref, v2_ref, haug_ref, a_ref, b_ref):
    h = jnp.dot(x_ref[...], w_ref[...], preferred_element_type=jnp.float32)
    haug_ref[...] = _ones_aug(h, h.shape[0])
    a_ref[...] = (0.5 * jnp.dot(h, v1_ref[...],
                                preferred_element_type=jnp.float32)
                  ).astype(jnp.bfloat16)
    b_ref[...] = (0.5 * jnp.dot(h, v2_ref[...],
                                preferred_element_type=jnp.float32)
                  ).astype(jnp.bfloat16)


def _proj(X, W1, v11, v12):
    k_in, k_out = W1.shape
    tm = 512
    return pl.pallas_call(
        _proj_kernel,
        grid=(_N // tm,),
        in_specs=[
            pl.BlockSpec((tm, k_in), lambda i: (i, 0)),
            pl.BlockSpec((k_in, k_out), lambda i: (0, 0)),
            pl.BlockSpec((k_out, 1), lambda i: (0, 0)),
            pl.BlockSpec((k_out, 1), lambda i: (0, 0)),
        ],
        out_specs=[
            pl.BlockSpec((tm, k_out + 1), lambda i: (i, 0)),
            pl.BlockSpec((tm, 1), lambda i: (i, 0)),
            pl.BlockSpec((tm, 1), lambda i: (i, 0)),
        ],
        out_shape=[
            jax.ShapeDtypeStruct((_N, k_out + 1), jnp.bfloat16),
            jax.ShapeDtypeStruct((_N, 1), jnp.bfloat16),
            jax.ShapeDtypeStruct((_N, 1), jnp.bfloat16),
        ],
        compiler_params=pltpu.CompilerParams(dimension_semantics=("parallel",)),
    )(X, W1, v11, v12)


def _recip_r(oa, k):
    r = oa[:, k:k + 1]
    return r, 1.0 / jnp.where(r == 0.0, 1.0, r)


def _gat_enc1_kernel(a_ref, b_ref, A_ref, haug_ref, w_ref, v1_ref, v2_ref,
                     S_ref, r_ref, haug_out_ref, aout_ref, bout_ref):
    k = haug_ref.shape[1] - 1
    sb = _unnorm_attn(A_ref[...], a_ref[...], b_ref[...])
    S_ref[...] = sb
    oa = jnp.dot(sb, haug_ref[...], preferred_element_type=jnp.float32)
    r, rinv = _recip_r(oa, k)
    r_ref[...] = r
    o = oa[:, :k] * rinv
    ho = jnp.dot(o, w_ref[...], preferred_element_type=jnp.float32)
    haug_out_ref[...] = _ones_aug(ho, ho.shape[0])
    aout_ref[...] = (0.5 * jnp.dot(ho, v1_ref[...],
                                   preferred_element_type=jnp.float32)
                     ).astype(jnp.bfloat16)
    bout_ref[...] = (0.5 * jnp.dot(ho, v2_ref[...],
                                   preferred_element_type=jnp.float32)
                     ).astype(jnp.bfloat16)


def _gat_enc1(A, a, b, Haug, W, v1, v2):
    ka = Haug.shape[1]
    k2 = W.shape[1]
    return pl.pallas_call(
        _gat_enc1_kernel,
        grid=(_N // _TM,),
        in_specs=[
            pl.BlockSpec((_TM, 1), lambda i: (i, 0)),
            pl.BlockSpec((1, _N), lambda i: (0, 0)),
            pl.BlockSpec((_TM, _N), lambda i: (i, 0)),
            pl.BlockSpec((_N, ka), lambda i: (0, 0)),
            pl.BlockSpec((ka - 1, k2), lambda i: (0, 0)),
            pl.BlockSpec((k2, 1), lambda i: (0, 0)),
            pl.BlockSpec((k2, 1), lambda i: (0, 0)),
        ],
        out_specs=[
            pl.BlockSpec((_TM, _N), lambda i: (i, 0)),
            pl.BlockSpec((_TM, 1), lambda i: (i, 0)),
            pl.BlockSpec((_TM, k2 + 1), lambda i: (i, 0)),
            pl.BlockSpec((_TM, 1), lambda i: (i, 0)),
            pl.BlockSpec((_TM, 1), lambda i: (i, 0)),
        ],
        out_shape=[
            jax.ShapeDtypeStruct((_N, _N), jnp.bfloat16),
            jax.ShapeDtypeStruct((_N, 1), jnp.float32),
            jax.ShapeDtypeStruct((_N, k2 + 1), jnp.bfloat16),
            jax.ShapeDtypeStruct((_N, 1), jnp.bfloat16),
            jax.ShapeDtypeStruct((_N, 1), jnp.bfloat16),
        ],
        compiler_params=pltpu.CompilerParams(dimension_semantics=("parallel",)),
    )(a, b, A, Haug, W, v1, v2)


def _gat_enc2_kernel(a_ref, b_ref, A_ref, haug_ref, w_ref,
                     S_ref, r_ref, emb_ref, d_ref):
    k = haug_ref.shape[1] - 1
    sb = _unnorm_attn(A_ref[...], a_ref[...], b_ref[...])
    S_ref[...] = sb
    oa = jnp.dot(sb, haug_ref[...], preferred_element_type=jnp.float32)
    r, rinv = _recip_r(oa, k)
    r_ref[...] = r
    o = oa[:, :k] * rinv
    emb_ref[...] = o
    d = jax.lax.dot_general(o, w_ref[...], (((1,), (1,)), ((), ())),
                            preferred_element_type=jnp.float32)
    d_ref[...] = d.astype(jnp.bfloat16)


def _gat_enc2(A, a, b, Haug, W):
    ka = Haug.shape[1]
    k2 = W.shape[0]
    return pl.pallas_call(
        _gat_enc2_kernel,
        grid=(_N // _TM,),
        in_specs=[
            pl.BlockSpec((_TM, 1), lambda i: (i, 0)),
            pl.BlockSpec((1, _N), lambda i: (0, 0)),
            pl.BlockSpec((_TM, _N), lambda i: (i, 0)),
            pl.BlockSpec((_N, ka), lambda i: (0, 0)),
            pl.BlockSpec((k2, ka - 1), lambda i: (0, 0)),
        ],
        out_specs=[
            pl.BlockSpec((_TM, _N), lambda i: (i, 0)),
            pl.BlockSpec((_TM, 1), lambda i: (i, 0)),
            pl.BlockSpec((_TM, ka - 1), lambda i: (i, 0)),
            pl.BlockSpec((_TM, k2), lambda i: (i, 0)),
        ],
        out_shape=[
            jax.ShapeDtypeStruct((_N, _N), jnp.bfloat16),
            jax.ShapeDtypeStruct((_N, 1), jnp.float32),
            jax.ShapeDtypeStruct((_N, ka - 1), jnp.float32),
            jax.ShapeDtypeStruct((_N, k2), jnp.bfloat16),
        ],
        compiler_params=pltpu.CompilerParams(dimension_semantics=("parallel",)),
    )(a, b, A, Haug, W)


def _gat_dec_kernel(with_w, S_ref, r_ref, h_ref, *rest):
    if with_w:
        w_ref, out_ref = rest
    else:
        out_ref, = rest
    o = jnp.dot(S_ref[...], h_ref[...], preferred_element_type=jnp.float32)
    r = r_ref[...]
    o = o * (1.0 / jnp.where(r == 0.0, 1.0, r))
    if with_w:
        out_ref[...] = jax.lax.dot_general(
            o, w_ref[...], (((1,), (1,)), ((), ())),
            preferred_element_type=jnp.float32)
    else:
        out_ref[...] = o.astype(out_ref.dtype)


def _gat_dec(S, r, H, W=None, out_dtype=jnp.float32):
    k = H.shape[1]
    kout = W.shape[0] if W is not None else k
    in_specs = [
        pl.BlockSpec((_TM, _N), lambda i: (i, 0)),
        pl.BlockSpec((_TM, 1), lambda i: (i, 0)),
        pl.BlockSpec((_N, k), lambda i: (0, 0)),
    ]
    args = [S, r, H]
    if W is not None:
        in_specs.append(pl.BlockSpec((kout, k), lambda i: (0, 0)))
        args.append(W)
        out_dtype = jnp.float32
    body = functools.partial(_gat_dec_kernel, W is not None)
    return pl.pallas_call(
        body,
        grid=(_N // _TM,),
        in_specs=in_specs,
        out_specs=pl.BlockSpec((_TM, kout), lambda i: (i, 0)),
        out_shape=jax.ShapeDtypeStruct((_N, kout), out_dtype),
        compiler_params=pltpu.CompilerParams(dimension_semantics=("parallel",)),
    )(*args)


def _final_kernel(h1_ref, h2_ref, mu_ref, hf_ref, q_ref):
    hf = h1_ref[...] + _BETA * h2_ref[...]
    hf_ref[...] = hf
    mu = mu_ref[...]
    hn = jnp.sum(hf * hf, axis=1, keepdims=True)
    mn = jnp.sum(mu * mu, axis=1)[None, :]
    cross = jax.lax.dot_general(hf, mu, (((1,), (1,)), ((), ())),
                                preferred_element_type=jnp.float32)
    d2 = hn + mn - 2.0 * cross
    qun = (1.0 + d2 / _ALPHA) ** (-(_ALPHA + 1.0) / 2.0)
    q_ref[...] = qun / jnp.sum(qun, axis=1, keepdims=True)


def _final(H1, H2, mu):
    k = H1.shape[1]
    nc = mu.shape[0]
    tm = 512
    return pl.pallas_call(
        _final_kernel,
        grid=(_N // tm,),
        in_specs=[
            pl.BlockSpec((tm, k), lambda i: (i, 0)),
            pl.BlockSpec((tm, k), lambda i: (i, 0)),
            pl.BlockSpec((nc, k), lambda i: (0, 0)),
        ],
        out_specs=[
            pl.BlockSpec((tm, k), lambda i: (i, 0)),
            pl.BlockSpec((tm, nc), lambda i: (i, 0)),
        ],
        out_shape=[
            jax.ShapeDtypeStruct((_N, k), jnp.float32),
            jax.ShapeDtypeStruct((_N, nc), jnp.float32),
        ],
        compiler_params=pltpu.CompilerParams(dimension_semantics=("parallel",)),
    )(H1, H2, mu)


def _branch(A, X, W1, v11, v12, W2, v21, v22):
    H0aug, a0, b0 = _proj(X, W1, v11, v12)
    S0, r0, H1aug, a1, b1 = _gat_enc1(A, a0, b0.reshape(1, _N), H0aug,
                                      W2, v21, v22)
    S1, r1, emb, D1 = _gat_enc2(A, a1, b1.reshape(1, _N), H1aug, W2)
    T = _gat_dec(S1, r1, D1, out_dtype=jnp.bfloat16)
    X_ = _gat_dec(S0, r0, T, W1)
    return emb, X_


def kernel(A, X, A2, X2, W11, v111, v112, W12, v121, v122, W21, v211, v212,
           W22, v221, v222, mu):
    H1, X_ = _branch(A, X, W11, v111, v112, W12, v121, v122)
    H2, X_2 = _branch(A2, X2, W21, v211, v212, W22, v221, v222)
    H_F, q = _final(H1, H2, mu)
    return (H_F, q, H1, H2, X_, X_2)


# speedup vs baseline: 2.0623x; 1.1088x over previous
import functools

import jax
import jax.numpy as jnp
from jax.experimental import pallas as pl
from jax.experimental.pallas import tpu as pltpu

_N = 4096
_TM = 512
_GS = _N // _TM
_BETA = 0.5
_ALPHA = 1.0
_C = 0.72134752044448170367996234050095


def _unnorm_attn(A_t, ah_col, bh_row):
    half = A_t.astype(jnp.bfloat16) * (ah_col + bh_row)
    cb = jnp.bfloat16(_C)
    s = jnp.exp2(cb * jnp.tanh(half) + cb)
    return jnp.where(half != jnp.bfloat16(0.0), s, jnp.bfloat16(0.0))


def _ones_aug(h, tm):
    return jnp.concatenate([h, jnp.ones((tm, 1), jnp.float32)],
                           axis=1).astype(jnp.bfloat16)


def _recip_r(oa, k):
    r = oa[:, k:k + 1]
    return r, 1.0 / jnp.where(r == 0.0, 1.0, r)


def _b_row(v_ref, haug, k):
    b = jax.lax.dot_general(v_ref[...].astype(jnp.bfloat16), haug[:, :k],
                            (((0,), (1,)), ((), ())),
                            preferred_element_type=jnp.float32)
    return (0.5 * b).astype(jnp.bfloat16)


def _enc_kernel(x_ref, w1_ref, v11_ref, v12_ref, a_ref, w2_ref, v21_ref,
                v22_ref, s0_ref, r0_ref, s1_ref, r1_ref, emb_ref, d1_ref,
                haug0_scr, a0_scr, haug1_scr, a1_scr):
    i = pl.program_id(1)
    k0 = haug0_scr.shape[1] - 1
    k1 = haug1_scr.shape[1] - 1
    rows = pl.ds(i * _TM, _TM)

    @pl.when(pl.program_id(0) == 0)
    def _proj():
        h = jnp.dot(x_ref[...], w1_ref[...],
                    preferred_element_type=jnp.float32)
        haug0_scr[rows, :] = _ones_aug(h, _TM)
        a0_scr[rows, :] = 0.5 * jnp.dot(h, v11_ref[...],
                                        preferred_element_type=jnp.float32)

    @pl.when(pl.program_id(0) == 1)
    def _enc1():
        haug0 = haug0_scr[...]
        sb = _unnorm_attn(a_ref[...], a0_scr[rows, :].astype(jnp.bfloat16),
                          _b_row(v12_ref, haug0, k0))
        s0_ref[...] = sb
        oa = jnp.dot(sb, haug0, preferred_element_type=jnp.float32)
        r, rinv = _recip_r(oa, k0)
        r0_ref[...] = r
        ho = jnp.dot(oa[:, :k0] * rinv, w2_ref[...],
                     preferred_element_type=jnp.float32)
        haug1_scr[rows, :] = _ones_aug(ho, _TM)
        a1_scr[rows, :] = 0.5 * jnp.dot(ho, v21_ref[...],
                                        preferred_element_type=jnp.float32)

    @pl.when(pl.program_id(0) == 2)
    def _enc2():
        haug1 = haug1_scr[...]
        sb = _unnorm_attn(a_ref[...], a1_scr[rows, :].astype(jnp.bfloat16),
                          _b_row(v22_ref, haug1, k1))
        s1_ref[...] = sb
        oa = jnp.dot(sb, haug1, preferred_element_type=jnp.float32)
        r, rinv = _recip_r(oa, k1)
        r1_ref[...] = r
        o = oa[:, :k1] * rinv
        emb_ref[...] = o
        d = jax.lax.dot_general(o, w2_ref[...], (((1,), (1,)), ((), ())),
                                preferred_element_type=jnp.float32)
        d1_ref[...] = d.astype(jnp.bfloat16)


def _enc(A, X, W1, v11, v12, W2, v21, v22):
    k_in, k0 = W1.shape
    k1 = W2.shape[1]
    return pl.pallas_call(
        _enc_kernel,
        grid=(3, _GS),
        in_specs=[
            pl.BlockSpec((_TM, k_in),
                         lambda p, i: (jnp.where(p == 0, i, 0), 0)),
            pl.BlockSpec((k_in, k0), lambda p, i: (0, 0)),
            pl.BlockSpec((k0, 1), lambda p, i: (0, 0)),
            pl.BlockSpec((k0, 1), lambda p, i: (0, 0)),
            pl.BlockSpec((_TM, _N),
                         lambda p, i: (jnp.where(p == 0, 0, i), 0)),
            pl.BlockSpec((k0, k1), lambda p, i: (0, 0)),
            pl.BlockSpec((k1, 1), lambda p, i: (0, 0)),
            pl.BlockSpec((k1, 1), lambda p, i: (0, 0)),
        ],
        out_specs=[
            pl.BlockSpec((_TM, _N), lambda p, i: (
                jnp.where(p == 1, i, jnp.where(p == 0, 0, _GS - 1)), 0)),
            pl.BlockSpec((_TM, 1), lambda p, i: (
                jnp.where(p == 1, i, jnp.where(p == 0, 0, _GS - 1)), 0)),
            pl.BlockSpec((_TM, _N),
                         lambda p, i: (jnp.where(p == 2, i, 0), 0)),
            pl.BlockSpec((_TM, 1),
                         lambda p, i: (jnp.where(p == 2, i, 0), 0)),
            pl.BlockSpec((_TM, k1),
                         lambda p, i: (jnp.where(p == 2, i, 0), 0)),
            pl.BlockSpec((_TM, k0),
                         lambda p, i: (jnp.where(p == 2, i, 0), 0)),
        ],
        out_shape=[
            jax.ShapeDtypeStruct((_N, _N), jnp.bfloat16),
            jax.ShapeDtypeStruct((_N, 1), jnp.float32),
            jax.ShapeDtypeStruct((_N, _N), jnp.bfloat16),
            jax.ShapeDtypeStruct((_N, 1), jnp.float32),
            jax.ShapeDtypeStruct((_N, k1), jnp.float32),
            jax.ShapeDtypeStruct((_N, k0), jnp.bfloat16),
        ],
        scratch_shapes=[
            pltpu.VMEM((_N, k0 + 1), jnp.bfloat16),
            pltpu.VMEM((_N, 1), jnp.float32),
            pltpu.VMEM((_N, k1 + 1), jnp.bfloat16),
            pltpu.VMEM((_N, 1), jnp.float32),
        ],
        compiler_params=pltpu.CompilerParams(
            dimension_semantics=("arbitrary", "arbitrary")),
    )(X, W1, v11, v12, A, W2, v21, v22)


def _dec_kernel(with_head, s1_ref, r1_ref, d1_ref, s0_ref, r0_ref, w1_ref,
                *rest):
    if with_head:
        (h1_ref, emb_ref, mu_ref, x_ref, hf_ref, q_ref, t_scr) = rest
    else:
        x_ref, t_scr = rest
    i = pl.program_id(1)
    rows = pl.ds(i * _TM, _TM)

    @pl.when(pl.program_id(0) == 0)
    def _dec1():
        o = jnp.dot(s1_ref[...], d1_ref[...],
                    preferred_element_type=jnp.float32)
        r = r1_ref[...]
        o = o * (1.0 / jnp.where(r == 0.0, 1.0, r))
        t_scr[rows, :] = o.astype(jnp.bfloat16)

    @pl.when(pl.program_id(0) == 1)
    def _dec2():
        o = jnp.dot(s0_ref[...], t_scr[...],
                    preferred_element_type=jnp.float32)
        r = r0_ref[...]
        o = o * (1.0 / jnp.where(r == 0.0, 1.0, r))
        x_ref[...] = jax.lax.dot_general(o, w1_ref[...],
                                         (((1,), (1,)), ((), ())),
                                         preferred_element_type=jnp.float32)
        if with_head:
            hf = h1_ref[...] + _BETA * emb_ref[...]
            hf_ref[...] = hf
            mu = mu_ref[...]
            hn = jnp.sum(hf * hf, axis=1, keepdims=True)
            mn = jnp.sum(mu * mu, axis=1)[None, :]
            cross = jax.lax.dot_general(hf, mu, (((1,), (1,)), ((), ())),
                                        preferred_element_type=jnp.float32)
            d2 = hn + mn - 2.0 * cross
            qun = (1.0 + d2 / _ALPHA) ** (-(_ALPHA + 1.0) / 2.0)
            q_ref[...] = qun / jnp.sum(qun, axis=1, keepdims=True)


def _dec(S1, r1, D1, S0, r0, W1, head=None):
    k1 = D1.shape[1]
    kout = W1.shape[0]
    in_specs = [
        pl.BlockSpec((_TM, _N),
                     lambda p, i: (jnp.where(p == 0, i, _GS - 1), 0)),
        pl.BlockSpec((_TM, 1),
                     lambda p, i: (jnp.where(p == 0, i, _GS - 1), 0)),
        pl.BlockSpec((_N, k1), lambda p, i: (0, 0)),
        pl.BlockSpec((_TM, _N),
                     lambda p, i: (jnp.where(p == 1, i, 0), 0)),
        pl.BlockSpec((_TM, 1),
                     lambda p, i: (jnp.where(p == 1, i, 0), 0)),
        pl.BlockSpec((kout, k1), lambda p, i: (0, 0)),
    ]
    args = [S1, r1, D1, S0, r0, W1]
    out_specs = [pl.BlockSpec((_TM, kout),
                              lambda p, i: (jnp.where(p == 1, i, 0), 0))]
    out_shape = [jax.ShapeDtypeStruct((_N, kout), jnp.float32)]
    if head is not None:
        H1, emb, mu = head
        ke = emb.shape[1]
        nc = mu.shape[0]
        in_specs += [
            pl.BlockSpec((_TM, ke),
                         lambda p, i: (jnp.where(p == 1, i, 0), 0)),
            pl.BlockSpec((_TM, ke),
                         lambda p, i: (jnp.where(p == 1, i, 0), 0)),
            pl.BlockSpec((nc, ke), lambda p, i: (0, 0)),
        ]
        args += [H1, emb, mu]
        out_specs += [
            pl.BlockSpec((_TM, ke),
                         lambda p, i: (jnp.where(p == 1, i, 0), 0)),
            pl.BlockSpec((_TM, nc),
                         lambda p, i: (jnp.where(p == 1, i, 0), 0)),
        ]
        out_shape += [
            jax.ShapeDtypeStruct((_N, ke), jnp.float32),
            jax.ShapeDtypeStruct((_N, nc), jnp.float32),
        ]
    body = functools.partial(_dec_kernel, head is not None)
    return pl.pallas_call(
        body,
        grid=(2, _GS),
        in_specs=in_specs,
        out_specs=out_specs,
        out_shape=out_shape,
        scratch_shapes=[pltpu.VMEM((_N, k1), jnp.bfloat16)],
        compiler_params=pltpu.CompilerParams(
            dimension_semantics=("arbitrary", "arbitrary")),
    )(*args)


def kernel(A, X, A2, X2, W11, v111, v112, W12, v121, v122, W21, v211, v212,
           W22, v221, v222, mu):
    S0, r0, S1, r1, H1, D1 = _enc(A, X, W11, v111, v112, W12, v121, v122)
    S0b, r0b, S1b, r1b, H2, D1b = _enc(A2, X2, W21, v211, v212, W22, v221,
                                       v222)
    (X_,) = _dec(S1, r1, D1, S0, r0, W11)
    X_2, H_F, q = _dec(S1b, r1b, D1b, S0b, r0b, W21, head=(H1, H2, mu))
    return (H_F, q, H1, H2, X_, X_2)


# speedup vs baseline: 2.2429x; 1.0876x over previous
import functools

import jax
import jax.numpy as jnp
from jax.experimental import pallas as pl
from jax.experimental.pallas import tpu as pltpu

_N = 4096
_TM = 512
_GS = _N // _TM
_BETA = 0.5
_ALPHA = 1.0
_C = 0.72134752044448170367996234050095


def _unnorm_attn(ab_t, ah_col, bh_row):
    half = ab_t * (ah_col + bh_row)
    cb = jnp.bfloat16(_C)
    s = jnp.exp2(cb * jnp.tanh(half) + cb)
    return jnp.where(half != jnp.bfloat16(0.0), s, jnp.bfloat16(0.0))


def _ones_aug(h, tm):
    return jnp.concatenate([h, jnp.ones((tm, 1), jnp.float32)],
                           axis=1).astype(jnp.bfloat16)


def _recip_r(oa, k):
    r = oa[:, k:k + 1]
    return 1.0 / jnp.where(r == 0.0, 1.0, r)


def _b_row(v_ref, haug, k):
    b = jax.lax.dot_general(v_ref[...].astype(jnp.bfloat16), haug[:, :k],
                            (((0,), (1,)), ((), ())),
                            preferred_element_type=jnp.float32)
    return (0.5 * b).astype(jnp.bfloat16)


def _branch_kernel(with_head, x_ref, w1_ref, v11_ref, v12_ref, a_ref,
                   w2_ref, v21_ref, v22_ref, *rest):
    if with_head:
        (h1_ref, mu_ref, emb_ref, x_out_ref, hf_ref, q_ref,
         abf_scr, haug0_scr, a0_scr, haug1_scr, a1_scr, d1_scr,
         t_scr) = rest
    else:
        (emb_ref, x_out_ref,
         abf_scr, haug0_scr, a0_scr, haug1_scr, a1_scr, d1_scr,
         t_scr) = rest
    i = pl.program_id(1)
    k0 = haug0_scr.shape[1] - 1
    k1 = haug1_scr.shape[1] - 1
    rows = pl.ds(i * _TM, _TM)

    @pl.when(pl.program_id(0) == 0)
    def _proj():
        h = jnp.dot(x_ref[...], w1_ref[...],
                    preferred_element_type=jnp.float32)
        haug0_scr[rows, :] = _ones_aug(h, _TM)
        a0_scr[rows, :] = (0.5 * jnp.dot(h, v11_ref[...],
                                         preferred_element_type=jnp.float32)
                           ).astype(jnp.bfloat16)

    @pl.when(pl.program_id(0) == 1)
    def _enc1():
        ab_t = a_ref[...].astype(jnp.bfloat16)
        abf_scr[rows, :] = ab_t
        haug0 = haug0_scr[...]
        sb = _unnorm_attn(ab_t, a0_scr[rows, :], _b_row(v12_ref, haug0, k0))
        oa = jnp.dot(sb, haug0, preferred_element_type=jnp.float32)
        ho = jnp.dot(oa[:, :k0] * _recip_r(oa, k0), w2_ref[...],
                     preferred_element_type=jnp.float32)
        haug1_scr[rows, :] = _ones_aug(ho, _TM)
        a1_scr[rows, :] = (0.5 * jnp.dot(ho, v21_ref[...],
                                         preferred_element_type=jnp.float32)
                           ).astype(jnp.bfloat16)

    @pl.when(pl.program_id(0) == 2)
    def _enc2():
        haug1 = haug1_scr[...]
        sb = _unnorm_attn(abf_scr[rows, :], a1_scr[rows, :],
                          _b_row(v22_ref, haug1, k1))
        oa = jnp.dot(sb, haug1, preferred_element_type=jnp.float32)
        o = oa[:, :k1] * _recip_r(oa, k1)
        emb_ref[...] = o
        d = jax.lax.dot_general(o, w2_ref[...], (((1,), (1,)), ((), ())),
                                preferred_element_type=jnp.float32)
        d1_scr[rows, :k0] = d.astype(jnp.bfloat16)
        d1_scr[rows, k0:] = jnp.ones((_TM, 1), jnp.bfloat16)

    @pl.when(pl.program_id(0) == 3)
    def _dec1():
        haug1 = haug1_scr[...]
        sb = _unnorm_attn(abf_scr[rows, :], a1_scr[rows, :],
                          _b_row(v22_ref, haug1, k1))
        oa = jnp.dot(sb, d1_scr[...], preferred_element_type=jnp.float32)
        t = oa[:, :k0] * _recip_r(oa, k0)
        t_scr[rows, :k0] = t.astype(jnp.bfloat16)
        t_scr[rows, k0:] = jnp.ones((_TM, 1), jnp.bfloat16)

    @pl.when(pl.program_id(0) == 4)
    def _dec2():
        haug0 = haug0_scr[...]
        sb = _unnorm_attn(abf_scr[rows, :], a0_scr[rows, :],
                          _b_row(v12_ref, haug0, k0))
        oa = jnp.dot(sb, t_scr[...], preferred_element_type=jnp.float32)
        o = oa[:, :k0] * _recip_r(oa, k0)
        x_out_ref[...] = jax.lax.dot_general(
            o, w1_ref[...], (((1,), (1,)), ((), ())),
            preferred_element_type=jnp.float32)
        if with_head:
            hf = h1_ref[...] + _BETA * emb_ref[...]
            hf_ref[...] = hf
            mu = mu_ref[...]
            hn = jnp.sum(hf * hf, axis=1, keepdims=True)
            mn = jnp.sum(mu * mu, axis=1)[None, :]
            cross = jax.lax.dot_general(hf, mu, (((1,), (1,)), ((), ())),
                                        preferred_element_type=jnp.float32)
            d2 = hn + mn - 2.0 * cross
            qun = (1.0 + d2 / _ALPHA) ** (-(_ALPHA + 1.0) / 2.0)
            q_ref[...] = qun / jnp.sum(qun, axis=1, keepdims=True)


def _branch(A, X, W1, v11, v12, W2, v21, v22, head=None):
    k_in, k0 = W1.shape
    k1 = W2.shape[1]
    in_specs = [
        pl.BlockSpec((_TM, k_in),
                     lambda p, i: (jnp.where(p == 0, i, _GS - 1), 0)),
        pl.BlockSpec((k_in, k0), lambda p, i: (0, 0)),
        pl.BlockSpec((k0, 1), lambda p, i: (0, 0)),
        pl.BlockSpec((k0, 1), lambda p, i: (0, 0)),
        pl.BlockSpec((_TM, _N), lambda p, i: (
            jnp.where(p == 1, i, jnp.where(p == 0, 0, _GS - 1)), 0)),
        pl.BlockSpec((k0, k1), lambda p, i: (0, 0)),
        pl.BlockSpec((k1, 1), lambda p, i: (0, 0)),
        pl.BlockSpec((k1, 1), lambda p, i: (0, 0)),
    ]
    args = [X, W1, v11, v12, A, W2, v21, v22]
    out_specs = [
        pl.BlockSpec((_TM, k1), lambda p, i: (jnp.where(p == 2, i, 0), 0)),
        pl.BlockSpec((_TM, k_in),
                     lambda p, i: (jnp.where(p == 4, i, 0), 0)),
    ]
    out_shape = [
        jax.ShapeDtypeStruct((_N, k1), jnp.float32),
        jax.ShapeDtypeStruct((_N, k_in), jnp.float32),
    ]
    if head is not None:
        H1, mu = head
        nc = mu.shape[0]
        in_specs += [
            pl.BlockSpec((_TM, k1),
                         lambda p, i: (jnp.where(p == 4, i, 0), 0)),
            pl.BlockSpec((nc, k1), lambda p, i: (0, 0)),
        ]
        args += [H1, mu]
        out_specs += [
            pl.BlockSpec((_TM, k1),
                         lambda p, i: (jnp.where(p == 4, i, 0), 0)),
            pl.BlockSpec((_TM, nc),
                         lambda p, i: (jnp.where(p == 4, i, 0), 0)),
        ]
        out_shape += [
            jax.ShapeDtypeStruct((_N, k1), jnp.float32),
            jax.ShapeDtypeStruct((_N, nc), jnp.float32),
        ]
    body = functools.partial(_branch_kernel, head is not None)
    return pl.pallas_call(
        body,
        grid=(5, _GS),
        in_specs=in_specs,
        out_specs=out_specs,
        out_shape=out_shape,
        scratch_shapes=[
            pltpu.VMEM((_N, _N), jnp.bfloat16),
            pltpu.VMEM((_N, k0 + 1), jnp.bfloat16),
            pltpu.VMEM((_N, 1), jnp.bfloat16),
            pltpu.VMEM((_N, k1 + 1), jnp.bfloat16),
            pltpu.VMEM((_N, 1), jnp.bfloat16),
            pltpu.VMEM((_N, k0 + 1), jnp.bfloat16),
            pltpu.VMEM((_N, k0 + 1), jnp.bfloat16),
        ],
        compiler_params=pltpu.CompilerParams(
            dimension_semantics=("arbitrary", "arbitrary")),
    )(*args)


def kernel(A, X, A2, X2, W11, v111, v112, W12, v121, v122, W21, v211, v212,
           W22, v221, v222, mu):
    H1, X_ = _branch(A, X, W11, v111, v112, W12, v121, v122)
    H2, X_2, H_F, q = _branch(A2, X2, W21, v211, v212, W22, v221, v222,
                              head=(H1, mu))
    return (H_F, q, H1, H2, X_, X_2)


# speedup vs baseline: 2.2987x; 1.0249x over previous
import functools

import jax
import jax.numpy as jnp
from jax.experimental import pallas as pl
from jax.experimental.pallas import tpu as pltpu

_N = 4096
_TM = 512
_GS = _N // _TM
_BETA = 0.5
_ALPHA = 1.0
_C = 0.72134752044448170367996234050095


def _unnorm_attn(ab_t, ah_col, bh_row):
    half = ab_t * (ah_col + bh_row)
    cb = jnp.bfloat16(_C)
    s = jnp.exp2(cb * jnp.tanh(half) + cb)
    return jnp.where(half != jnp.bfloat16(0.0), s, jnp.bfloat16(0.0))


def _ones_aug(h, tm):
    return jnp.concatenate([h, jnp.ones((tm, 1), jnp.float32)],
                           axis=1).astype(jnp.bfloat16)


def _recip_r(oa, k):
    r = oa[:, k:k + 1]
    return 1.0 / jnp.where(r == 0.0, 1.0, r)


def _b_row(v_ref, haug, k):
    b = jax.lax.dot_general(v_ref[...].astype(jnp.bfloat16), haug[:, :k],
                            (((0,), (1,)), ((), ())),
                            preferred_element_type=jnp.float32)
    return (0.5 * b).astype(jnp.bfloat16)


def _branch_kernel(with_head, x_ref, w1_ref, v11_ref, v12_ref, a_ref,
                   w2_ref, v21_ref, v22_ref, *rest):
    if with_head:
        (h1_ref, mu_ref, emb_ref, x_out_ref, hf_ref, q_ref,
         abf_scr, haug0_scr, haug1_scr, av_scr, d1_scr,
         t_scr, b0r_scr, b1r_scr) = rest
    else:
        (emb_ref, x_out_ref,
         abf_scr, haug0_scr, haug1_scr, av_scr, d1_scr,
         t_scr, b0r_scr, b1r_scr) = rest
    i = pl.program_id(1)
    k0 = haug0_scr.shape[1] - 1
    k1 = haug1_scr.shape[1] - 1
    rows = pl.ds(i * _TM, _TM)

    @pl.when(pl.program_id(0) == 0)
    def _proj():
        h = jnp.dot(x_ref[...], w1_ref[...],
                    preferred_element_type=jnp.float32)
        haug0_scr[rows, :] = _ones_aug(h, _TM)
        av_scr[rows, 0:1] = (0.5 * jnp.dot(h, v11_ref[...],
                                           preferred_element_type=jnp.float32)
                             ).astype(jnp.bfloat16)

    @pl.when((pl.program_id(0) == 1) & (i == 0))
    def _b0():
        b0r_scr[...] = _b_row(v12_ref, haug0_scr[...], k0)

    @pl.when((pl.program_id(0) == 2) & (i == 0))
    def _b1():
        b1r_scr[...] = _b_row(v22_ref, haug1_scr[...], k1)

    @pl.when(pl.program_id(0) == 1)
    def _enc1():
        ab_t = a_ref[...].astype(jnp.bfloat16)
        abf_scr[rows, :] = ab_t
        haug0 = haug0_scr[...]
        sb = _unnorm_attn(ab_t, av_scr[rows, 0:1], b0r_scr[...])
        oa = jnp.dot(sb, haug0, preferred_element_type=jnp.float32)
        ho = jnp.dot(oa[:, :k0] * _recip_r(oa, k0), w2_ref[...],
                     preferred_element_type=jnp.float32)
        haug1_scr[rows, :] = _ones_aug(ho, _TM)
        av_scr[rows, 1:2] = (0.5 * jnp.dot(ho, v21_ref[...],
                                           preferred_element_type=jnp.float32)
                             ).astype(jnp.bfloat16)

    @pl.when(pl.program_id(0) == 2)
    def _enc2():
        haug1 = haug1_scr[...]
        sb = _unnorm_attn(abf_scr[rows, :], av_scr[rows, 1:2], b1r_scr[...])
        oa = jnp.dot(sb, haug1, preferred_element_type=jnp.float32)
        o = oa[:, :k1] * _recip_r(oa, k1)
        emb_ref[...] = o
        d = jax.lax.dot_general(o, w2_ref[...], (((1,), (1,)), ((), ())),
                                preferred_element_type=jnp.float32)
        d1_scr[rows, :k0] = d.astype(jnp.bfloat16)
        d1_scr[rows, k0:k0 + 1] = jnp.ones((_TM, 1), jnp.bfloat16)
        d1_scr[rows, k0 + 1:k0 + 1 + k1] = o.astype(jnp.bfloat16)

    @pl.when(pl.program_id(0) == 3)
    def _dec1():
        sb = _unnorm_attn(abf_scr[rows, :], av_scr[rows, 1:2], b1r_scr[...])
        oa = jnp.dot(sb, d1_scr[:, :k0 + 1],
                     preferred_element_type=jnp.float32)
        t = oa[:, :k0] * _recip_r(oa, k0)
        t_scr[rows, :k0] = t.astype(jnp.bfloat16)
        t_scr[rows, k0:] = jnp.ones((_TM, 1), jnp.bfloat16)

    @pl.when(pl.program_id(0) == 4)
    def _dec2():
        sb = _unnorm_attn(abf_scr[rows, :], av_scr[rows, 0:1], b0r_scr[...])
        oa = jnp.dot(sb, t_scr[...], preferred_element_type=jnp.float32)
        o = oa[:, :k0] * _recip_r(oa, k0)
        x_out_ref[...] = jax.lax.dot_general(
            o, w1_ref[...], (((1,), (1,)), ((), ())),
            preferred_element_type=jnp.float32)
        if with_head:
            hf = (h1_ref[...]
                  + _BETA * d1_scr[rows, k0 + 1:k0 + 1 + k1].astype(
                      jnp.float32))
            hf_ref[...] = hf
            mu = mu_ref[...]
            hn = jnp.sum(hf * hf, axis=1, keepdims=True)
            mn = jnp.sum(mu * mu, axis=1)[None, :]
            cross = jax.lax.dot_general(hf, mu, (((1,), (1,)), ((), ())),
                                        preferred_element_type=jnp.float32)
            d2 = hn + mn - 2.0 * cross
            qun = (1.0 + d2 / _ALPHA) ** (-(_ALPHA + 1.0) / 2.0)
            q_ref[...] = qun / jnp.sum(qun, axis=1, keepdims=True)


def _branch(A, X, W1, v11, v12, W2, v21, v22, head=None):
    k_in, k0 = W1.shape
    k1 = W2.shape[1]
    in_specs = [
        pl.BlockSpec((_TM, k_in),
                     lambda p, i: (jnp.where(p == 0, i, _GS - 1), 0)),
        pl.BlockSpec((k_in, k0), lambda p, i: (0, 0)),
        pl.BlockSpec((k0, 1), lambda p, i: (0, 0)),
        pl.BlockSpec((k0, 1), lambda p, i: (0, 0)),
        pl.BlockSpec((_TM, _N), lambda p, i: (
            jnp.where(p == 1, i, jnp.where(p == 0, 0, _GS - 1)), 0)),
        pl.BlockSpec((k0, k1), lambda p, i: (0, 0)),
        pl.BlockSpec((k1, 1), lambda p, i: (0, 0)),
        pl.BlockSpec((k1, 1), lambda p, i: (0, 0)),
    ]
    args = [X, W1, v11, v12, A, W2, v21, v22]
    out_specs = [
        pl.BlockSpec((_TM, k1), lambda p, i: (jnp.where(p == 2, i, 0), 0)),
        pl.BlockSpec((_TM, k_in),
                     lambda p, i: (jnp.where(p == 4, i, 0), 0)),
    ]
    out_shape = [
        jax.ShapeDtypeStruct((_N, k1), jnp.float32),
        jax.ShapeDtypeStruct((_N, k_in), jnp.float32),
    ]
    if head is not None:
        H1, mu = head
        nc = mu.shape[0]
        in_specs += [
            pl.BlockSpec((_TM, k1),
                         lambda p, i: (jnp.where(p == 4, i, 0), 0)),
            pl.BlockSpec((nc, k1), lambda p, i: (0, 0)),
        ]
        args += [H1, mu]
        out_specs += [
            pl.BlockSpec((_TM, k1),
                         lambda p, i: (jnp.where(p == 4, i, 0), 0)),
            pl.BlockSpec((_TM, nc),
                         lambda p, i: (jnp.where(p == 4, i, 0), 0)),
        ]
        out_shape += [
            jax.ShapeDtypeStruct((_N, k1), jnp.float32),
            jax.ShapeDtypeStruct((_N, nc), jnp.float32),
        ]
    body = functools.partial(_branch_kernel, head is not None)
    return pl.pallas_call(
        body,
        grid=(5, _GS),
        in_specs=in_specs,
        out_specs=out_specs,
        out_shape=out_shape,
        scratch_shapes=[
            pltpu.VMEM((_N, _N), jnp.bfloat16),
            pltpu.VMEM((_N, k0 + 1), jnp.bfloat16),
            pltpu.VMEM((_N, k1 + 1), jnp.bfloat16),
            pltpu.VMEM((_N, 2), jnp.bfloat16),
            pltpu.VMEM((_N, k0 + 1 + k1), jnp.bfloat16),
            pltpu.VMEM((_N, k0 + 1), jnp.bfloat16),
            pltpu.VMEM((1, _N), jnp.bfloat16),
            pltpu.VMEM((1, _N), jnp.bfloat16),
        ],
        compiler_params=pltpu.CompilerParams(
            dimension_semantics=("arbitrary", "arbitrary")),
    )(*args)


def kernel(A, X, A2, X2, W11, v111, v112, W12, v121, v122, W21, v211, v212,
           W22, v221, v222, mu):
    H1, X_ = _branch(A, X, W11, v111, v112, W12, v121, v122)
    H2, X_2, H_F, q = _branch(A2, X2, W21, v211, v212, W22, v221, v222,
                              head=(H1, mu))
    return (H_F, q, H1, H2, X_, X_2)
